# Initial kernel scaffold; baseline (speedup 1.0000x reference)
import jax
import jax.numpy as jnp
from jax.experimental import pallas as pl


def kernel(temporal_features, sup, params):
    raise NotImplementedError("write your pallas kernel here")


# speedup vs baseline: 3.9355x; 3.9355x over previous
import jax
import jax.numpy as jnp
from jax.experimental import pallas as pl
from jax.experimental.pallas import tpu as pltpu

GROUPS = 16
EPS = 1e-5
_NB = 512


def _gelu(x):
    return 0.5 * x * (1.0 + jax.lax.erf(x * (2.0 ** -0.5)))


def _ssm_kernel(x_ref, sup_ref,
                in1, w11, b11, w12, b12, g1w, g1b, s1w, s1b, o1w,
                in2, w21, b21, w22, b22, g2w, g2b, s2w, s2b, o2w,
                o_ref):
    T, C, NB = x_ref.shape

    sup_rows = [sup_ref[t:t + 1, :] for t in range(T)]

    def dense(Xb, inw, w1, bias1, w2, bias2, gw, gbias):
        Xin = jnp.dot(inw[...], Xb, preferred_element_type=jnp.float32)
        hh = _gelu(jnp.dot(w1[...], Xin.astype(jnp.bfloat16),
                           preferred_element_type=jnp.float32) + b1_ld(bias1))
        h = jnp.dot(w2[...], hh.astype(jnp.bfloat16),
                    preferred_element_type=jnp.float32) + b1_ld(bias2)
        Xt = Xin + h
        G = jax.nn.sigmoid(jnp.dot(gw[...], Xt.astype(jnp.bfloat16),
                                   preferred_element_type=jnp.float32) + b1_ld(gbias))
        return Xt, G

    def b1_ld(ref):
        return ref[...]

    def scan(Xt, G, sw, sb, accumulate):
        s = jnp.zeros((C, NB), jnp.float32)
        outs = []
        ssum = jnp.zeros((C, NB), jnp.float32)
        swv = sw[...]
        sbv = sb[...]
        for t in range(T):
            g = G[:, t * NB:(t + 1) * NB] * jax.nn.sigmoid(swv * sup_rows[t] + sbv)
            x_t = Xt[:, t * NB:(t + 1) * NB]
            s = (1.0 - g) * s + g * x_t
            if accumulate:
                ssum = ssum + s
            else:
                outs.append(s)
        return outs, ssum

    X = jnp.concatenate(
        [x_ref[t] * (1.0 - sup_rows[t]) for t in range(T)], axis=1)

    Xt1, G1 = dense(X.astype(jnp.bfloat16), in1, w11, b11, w12, b12, g1w, g1b)
    S1, _ = scan(Xt1, G1, s1w, s1b, accumulate=False)
    S1cat = jnp.concatenate(S1, axis=1)
    X2 = jnp.dot(o1w[...], S1cat.astype(jnp.bfloat16),
                 preferred_element_type=jnp.float32)

    Xt2, G2 = dense(X2.astype(jnp.bfloat16), in2, w21, b21, w22, b22, g2w, g2b)
    _, ssum = scan(Xt2, G2, s2w, s2b, accumulate=True)

    o_ref[...] = jnp.dot(o2w[...], (ssum * (1.0 / T)).astype(jnp.bfloat16),
                         preferred_element_type=jnp.float32).astype(jnp.bfloat16)


def _conv_kernel(x_ref, w_ref, scale_ref, bias_ref, gmat_ref, res_ref, o_ref, W):
    C, HW = x_ref.shape
    xb = x_ref[...]
    p = jax.lax.broadcasted_iota(jnp.int32, (C, HW), 1)
    wv = p & (W - 1)

    Y = jnp.zeros((C, HW), jnp.float32)
    for dy in range(3):
        for dx in range(3):
            wk = w_ref[dy, dx]
            d = jnp.dot(wk, xb, preferred_element_type=jnp.float32)
            shift = (-(dy - 1) * W - (dx - 1)) % HW
            if shift != 0:
                d = pltpu.roll(d, shift, axis=1)
            cond = None
            if dy == 0:
                cond = p >= W
            elif dy == 2:
                cond = p < HW - W
            if dx == 0:
                c2 = wv >= 1
                cond = c2 if cond is None else (cond & c2)
            elif dx == 2:
                c2 = wv < W - 1
                cond = c2 if cond is None else (cond & c2)
            if cond is not None:
                d = jnp.where(cond, d, 0.0)
            Y = Y + d

    rsum = jnp.sum(Y, axis=1, keepdims=True)
    rsq = jnp.sum(Y * Y, axis=1, keepdims=True)
    gm = gmat_ref[...]
    mean = jnp.dot(gm, rsum, preferred_element_type=jnp.float32)
    msq = jnp.dot(gm, rsq, preferred_element_type=jnp.float32)
    var = msq - mean * mean
    rstd = jax.lax.rsqrt(var + EPS)
    yn = (Y - mean) * rstd * scale_ref[...] + bias_ref[...]
    act = _gelu(yn)
    if res_ref is not None:
        act = act + res_ref[...].astype(jnp.float32)
    o_ref[...] = act.astype(o_ref.dtype)


def _mk_conv_body(with_res, img_w):
    if with_res:
        def body(x_ref, w_ref, scale_ref, bias_ref, gmat_ref, res_ref, o_ref):
            _conv_kernel(x_ref, w_ref, scale_ref, bias_ref, gmat_ref, res_ref,
                         o_ref, img_w)
    else:
        def body(x_ref, w_ref, scale_ref, bias_ref, gmat_ref, o_ref):
            _conv_kernel(x_ref, w_ref, scale_ref, bias_ref, gmat_ref, None,
                         o_ref, img_w)
    return body


def _conv_stage(x_col, w4, scale, bias, gmat, img_w, img_h,
                res_col=None, out_f32=False):
    C, N = x_col.shape
    HW = img_h * img_w
    B = N // HW
    kb = pl.BlockSpec((C, HW), lambda b: (0, b))
    wspec = pl.BlockSpec(w4.shape, lambda b: (0, 0, 0, 0))
    cspec = pl.BlockSpec((C, 1), lambda b: (0, 0))
    gspec = pl.BlockSpec((C, C), lambda b: (0, 0))
    args = [x_col, w4, scale, bias, gmat]
    specs = [kb, wspec, cspec, cspec, gspec]
    if res_col is not None:
        args.append(res_col)
        specs.append(kb)
    return pl.pallas_call(
        _mk_conv_body(res_col is not None, img_w),
        out_shape=jax.ShapeDtypeStruct((C, N), jnp.float32 if out_f32 else jnp.bfloat16),
        grid=(B,),
        in_specs=specs,
        out_specs=kb,
        compiler_params=pltpu.CompilerParams(
            dimension_semantics=("arbitrary",),
            vmem_limit_bytes=100 * 1024 * 1024,
        ),
        name="conv_gn_act",
    )(*args)


def kernel(temporal_features, sup, params):
    B, T, C, H, W = temporal_features.shape
    N = B * H * W
    HID = params['blocks'][0]['ffn_w1'].shape[0]

    sup_hw = jax.image.resize(sup, (B, T, H, W), 'bilinear')
    x_cm = temporal_features.transpose(1, 2, 0, 3, 4).reshape(T, C, N)
    sup_tn = sup_hw.transpose(1, 0, 2, 3).reshape(T, N)

    bf = lambda a: a.astype(jnp.bfloat16)
    col = lambda v: v.reshape(-1, 1).astype(jnp.float32)
    ws = []
    for p in params['blocks']:
        ws += [bf(p['in_w']), bf(p['ffn_w1']), col(p['ffn_b1']),
               bf(p['ffn_w2']), col(p['ffn_b2']),
               bf(p['gate_w']), col(p['gate_b']),
               col(p['sup_w'][:, 0]), col(p['sup_b']), bf(p['out_w'])]

    nb = min(_NB, N)
    nblk = N // nb
    wspecs = [pl.BlockSpec(w.shape, lambda i: (0, 0)) for w in ws]

    y_col = pl.pallas_call(
        _ssm_kernel,
        out_shape=jax.ShapeDtypeStruct((C, N), jnp.bfloat16),
        grid=(nblk,),
        in_specs=[
            pl.BlockSpec((T, C, nb), lambda i: (0, 0, i)),
            pl.BlockSpec((T, nb), lambda i: (0, i)),
        ] + wspecs,
        out_specs=pl.BlockSpec((C, nb), lambda i: (0, i)),
        compiler_params=pltpu.CompilerParams(
            dimension_semantics=("arbitrary",),
            vmem_limit_bytes=100 * 1024 * 1024,
        ),
        name="ssm_fused",
    )(x_cm, sup_tn, *ws)

    cg = C // GROUPS
    gi = jnp.arange(C) // cg
    gmat = (gi[:, None] == gi[None, :]).astype(jnp.float32) / (H * W * cg)

    y1 = _conv_stage(y_col, bf(params['post_conv_w'].transpose(2, 3, 0, 1)),
                     col(params['post_gn_s']), col(params['post_gn_b']), gmat,
                     W, H)
    r1 = _conv_stage(y1, bf(params['rc1_w'].transpose(2, 3, 0, 1)),
                     col(params['rg1_s']), col(params['rg1_b']), gmat, W, H)
    out_col = _conv_stage(r1, bf(params['rc2_w'].transpose(2, 3, 0, 1)),
                          col(params['rg2_s']), col(params['rg2_b']), gmat,
                          W, H, res_col=y1)

    return out_col.reshape(C, B, H, W).transpose(1, 0, 2, 3).astype(jnp.float32)


# speedup vs baseline: 4.4293x; 1.1255x over previous
import jax
import jax.numpy as jnp
from jax.experimental import pallas as pl
from jax.experimental.pallas import tpu as pltpu

GROUPS = 16
EPS = 1e-5
_NB = 512


def _gelu(x):
    return 0.5 * x * (1.0 + jax.lax.erf(x * (2.0 ** -0.5)))


def _ssm_kernel(x_ref, sup_ref,
                in1, w11, b11, w12, b12, g1w, g1b, s1w, s1b, o1w,
                in2, w21, b21, w22, b22, g2w, g2b, s2w, s2b, o2w,
                o_ref):
    T, C, NB = x_ref.shape

    sup_rows = [sup_ref[t:t + 1, :] for t in range(T)]

    def dense(Xb, inw, w1, bias1, w2, bias2, gw, gbias):
        Xin = jnp.dot(inw[...], Xb, preferred_element_type=jnp.float32)
        hh = _gelu(jnp.dot(w1[...], Xin.astype(jnp.bfloat16),
                           preferred_element_type=jnp.float32) + b1_ld(bias1))
        h = jnp.dot(w2[...], hh.astype(jnp.bfloat16),
                    preferred_element_type=jnp.float32) + b1_ld(bias2)
        Xt = Xin + h
        G = jax.nn.sigmoid(jnp.dot(gw[...], Xt.astype(jnp.bfloat16),
                                   preferred_element_type=jnp.float32) + b1_ld(gbias))
        return Xt, G

    def b1_ld(ref):
        return ref[...]

    def scan(Xt, G, sw, sb, accumulate):
        s = jnp.zeros((C, NB), jnp.float32)
        outs = []
        ssum = jnp.zeros((C, NB), jnp.float32)
        swv = sw[...]
        sbv = sb[...]
        sup_cat = jnp.concatenate(sup_rows, axis=1)
        Gs = G * jax.nn.sigmoid(swv * sup_cat + sbv)
        for t in range(T):
            g = Gs[:, t * NB:(t + 1) * NB]
            x_t = Xt[:, t * NB:(t + 1) * NB]
            s = s + g * (x_t - s)
            if accumulate:
                ssum = ssum + s
            else:
                outs.append(s)
        return outs, ssum

    X = jnp.concatenate(
        [x_ref[t] * (1.0 - sup_rows[t]) for t in range(T)], axis=1)

    Xt1, G1 = dense(X.astype(jnp.bfloat16), in1, w11, b11, w12, b12, g1w, g1b)
    S1, _ = scan(Xt1, G1, s1w, s1b, accumulate=False)
    S1cat = jnp.concatenate(S1, axis=1)
    X2 = jnp.dot(o1w[...], S1cat.astype(jnp.bfloat16),
                 preferred_element_type=jnp.float32)

    Xt2, G2 = dense(X2.astype(jnp.bfloat16), in2, w21, b21, w22, b22, g2w, g2b)
    _, ssum = scan(Xt2, G2, s2w, s2b, accumulate=True)

    o_ref[...] = jnp.dot(o2w[...], (ssum * (1.0 / T)).astype(jnp.bfloat16),
                         preferred_element_type=jnp.float32).astype(jnp.bfloat16)


def _conv_kernel(x_ref, w_ref, scale_ref, bias_ref, gmat_ref, res_ref, o_ref, W):
    C, HW = x_ref.shape
    xb = x_ref[...]
    p = jax.lax.broadcasted_iota(jnp.int32, (C, HW), 1)
    wv = p & (W - 1)

    x32 = xb.astype(jnp.float32)
    xm = jnp.where(wv == 0, 0.0, pltpu.roll(x32, 1, axis=1)).astype(jnp.bfloat16)
    xp = jnp.where(wv == W - 1, 0.0,
                   pltpu.roll(x32, HW - 1, axis=1)).astype(jnp.bfloat16)
    im3 = jnp.concatenate([xm, xb, xp], axis=0)

    Y = jnp.dot(w_ref[1], im3, preferred_element_type=jnp.float32)
    d0 = jnp.dot(w_ref[0], im3, preferred_element_type=jnp.float32)
    Y = Y + jnp.where(p >= W, pltpu.roll(d0, W, axis=1), 0.0)
    d2 = jnp.dot(w_ref[2], im3, preferred_element_type=jnp.float32)
    Y = Y + jnp.where(p < HW - W, pltpu.roll(d2, HW - W, axis=1), 0.0)

    rsum = jnp.sum(Y, axis=1, keepdims=True)
    rsq = jnp.sum(Y * Y, axis=1, keepdims=True)
    gm = gmat_ref[...]
    mean = jnp.dot(gm, rsum, preferred_element_type=jnp.float32)
    msq = jnp.dot(gm, rsq, preferred_element_type=jnp.float32)
    var = msq - mean * mean
    a = jax.lax.rsqrt(var + EPS) * scale_ref[...]
    b = bias_ref[...] - mean * a
    act = _gelu(Y * a + b)
    if res_ref is not None:
        act = act + res_ref[...].astype(jnp.float32)
    o_ref[...] = act.astype(o_ref.dtype)


def _mk_conv_body(with_res, img_w):
    if with_res:
        def body(x_ref, w_ref, scale_ref, bias_ref, gmat_ref, res_ref, o_ref):
            _conv_kernel(x_ref, w_ref, scale_ref, bias_ref, gmat_ref, res_ref,
                         o_ref, img_w)
    else:
        def body(x_ref, w_ref, scale_ref, bias_ref, gmat_ref, o_ref):
            _conv_kernel(x_ref, w_ref, scale_ref, bias_ref, gmat_ref, None,
                         o_ref, img_w)
    return body


def _conv_stage(x_col, w4, scale, bias, gmat, img_w, img_h,
                res_col=None, out_f32=False):
    C, N = x_col.shape
    HW = img_h * img_w
    B = N // HW
    kb = pl.BlockSpec((C, HW), lambda b: (0, b))
    wspec = pl.BlockSpec(w4.shape, lambda b: (0, 0, 0))
    cspec = pl.BlockSpec((C, 1), lambda b: (0, 0))
    gspec = pl.BlockSpec((C, C), lambda b: (0, 0))
    args = [x_col, w4, scale, bias, gmat]
    specs = [kb, wspec, cspec, cspec, gspec]
    if res_col is not None:
        args.append(res_col)
        specs.append(kb)
    return pl.pallas_call(
        _mk_conv_body(res_col is not None, img_w),
        out_shape=jax.ShapeDtypeStruct((C, N), jnp.float32 if out_f32 else jnp.bfloat16),
        grid=(B,),
        in_specs=specs,
        out_specs=kb,
        compiler_params=pltpu.CompilerParams(
            dimension_semantics=("arbitrary",),
            vmem_limit_bytes=100 * 1024 * 1024,
        ),
        name="conv_gn_act",
    )(*args)


def kernel(temporal_features, sup, params):
    B, T, C, H, W = temporal_features.shape
    N = B * H * W
    HID = params['blocks'][0]['ffn_w1'].shape[0]

    sup_hw = jax.image.resize(sup, (B, T, H, W), 'bilinear')
    x_cm = temporal_features.transpose(1, 2, 0, 3, 4).reshape(T, C, N)
    sup_tn = sup_hw.transpose(1, 0, 2, 3).reshape(T, N)

    bf = lambda a: a.astype(jnp.bfloat16)
    col = lambda v: v.reshape(-1, 1).astype(jnp.float32)
    ws = []
    for p in params['blocks']:
        ws += [bf(p['in_w']), bf(p['ffn_w1']), col(p['ffn_b1']),
               bf(p['ffn_w2']), col(p['ffn_b2']),
               bf(p['gate_w']), col(p['gate_b']),
               col(p['sup_w'][:, 0]), col(p['sup_b']), bf(p['out_w'])]

    nb = min(_NB, N)
    nblk = N // nb
    ncore = 2 if nblk % 2 == 0 else 1
    nseq = nblk // ncore
    wspecs = [pl.BlockSpec(w.shape, lambda i, j: (0, 0)) for w in ws]

    y_col = pl.pallas_call(
        _ssm_kernel,
        out_shape=jax.ShapeDtypeStruct((C, N), jnp.bfloat16),
        grid=(ncore, nseq),
        in_specs=[
            pl.BlockSpec((T, C, nb), lambda i, j: (0, 0, i * nseq + j)),
            pl.BlockSpec((T, nb), lambda i, j: (0, i * nseq + j)),
        ] + wspecs,
        out_specs=pl.BlockSpec((C, nb), lambda i, j: (0, i * nseq + j)),
        compiler_params=pltpu.CompilerParams(
            dimension_semantics=("arbitrary", "arbitrary"),
            vmem_limit_bytes=100 * 1024 * 1024,
        ),
        name="ssm_fused",
    )(x_cm, sup_tn, *ws)

    cg = C // GROUPS
    gi = jnp.arange(C) // cg
    gmat = (gi[:, None] == gi[None, :]).astype(jnp.float32) / (H * W * cg)

    y1 = _conv_stage(y_col, bf(params['post_conv_w'].transpose(2, 0, 3, 1).reshape(3, C, 3 * C)),
                     col(params['post_gn_s']), col(params['post_gn_b']), gmat,
                     W, H)
    r1 = _conv_stage(y1, bf(params['rc1_w'].transpose(2, 0, 3, 1).reshape(3, C, 3 * C)),
                     col(params['rg1_s']), col(params['rg1_b']), gmat, W, H)
    out_col = _conv_stage(r1, bf(params['rc2_w'].transpose(2, 0, 3, 1).reshape(3, C, 3 * C)),
                          col(params['rg2_s']), col(params['rg2_b']), gmat,
                          W, H, res_col=y1)

    return out_col.reshape(C, B, H, W).transpose(1, 0, 2, 3).astype(jnp.float32)


# speedup vs baseline: 4.4654x; 1.0081x over previous
import jax
import jax.numpy as jnp
from jax.experimental import pallas as pl
from jax.experimental.pallas import tpu as pltpu

GROUPS = 16
EPS = 1e-5
_NB = 512


def _gelu(x):
    return 0.5 * x * (1.0 + jax.lax.erf(x * (2.0 ** -0.5)))


def _ssm_kernel(x_ref, sup_ref,
                in1, w11, b11, w12, b12, g1w, g1b, s1w, s1b, o1w,
                w21, b21, w22, b22, g2w, g2b, s2w, s2b, o2w,
                o_ref):
    T, C, NB = x_ref.shape

    sup_rows = [sup_ref[t:t + 1, :] for t in range(T)]

    def dense(Xb, inw, w1, bias1, w2, bias2, gw, gbias, colscale=None):
        Xin = jnp.dot(inw[...], Xb, preferred_element_type=jnp.float32)
        if colscale is not None:
            Xin = Xin * colscale
        hh = _gelu(jnp.dot(w1[...], Xin.astype(jnp.bfloat16),
                           preferred_element_type=jnp.float32) + b1_ld(bias1))
        h = jnp.dot(w2[...], hh.astype(jnp.bfloat16),
                    preferred_element_type=jnp.float32) + b1_ld(bias2)
        Xt = Xin + h
        G = jax.nn.sigmoid(jnp.dot(gw[...], Xt.astype(jnp.bfloat16),
                                   preferred_element_type=jnp.float32) + b1_ld(gbias))
        return Xt, G

    def b1_ld(ref):
        return ref[...]

    def scan(Xt, G, sw, sb, accumulate):
        s = jnp.zeros((C, NB), jnp.float32)
        outs = []
        ssum = jnp.zeros((C, NB), jnp.float32)
        swv = sw[...]
        sbv = sb[...]
        sup_cat = jnp.concatenate(sup_rows, axis=1)
        Gs = G * jax.nn.sigmoid(swv * sup_cat + sbv)
        for t in range(T):
            g = Gs[:, t * NB:(t + 1) * NB]
            x_t = Xt[:, t * NB:(t + 1) * NB]
            s = s + g * (x_t - s)
            if accumulate:
                ssum = ssum + s
            else:
                outs.append(s)
        return outs, ssum

    X = jnp.concatenate([x_ref[t] for t in range(T)], axis=1)
    fac = 1.0 - jnp.concatenate(sup_rows, axis=1)

    Xt1, G1 = dense(X, in1, w11, b11, w12, b12, g1w, g1b, colscale=fac)
    S1, _ = scan(Xt1, G1, s1w, s1b, accumulate=False)
    S1cat = jnp.concatenate(S1, axis=1)

    Xt2, G2 = dense(S1cat.astype(jnp.bfloat16), o1w, w21, b21, w22, b22,
                    g2w, g2b)
    _, ssum = scan(Xt2, G2, s2w, s2b, accumulate=True)

    o_ref[...] = jnp.dot(o2w[...], ssum.astype(jnp.bfloat16),
                         preferred_element_type=jnp.float32).astype(jnp.bfloat16)


def _conv_kernel(x_ref, w_ref, scale_ref, bias_ref, gmat_ref, res_ref, o_ref, W):
    C, HW = x_ref.shape
    xb = x_ref[...]
    p = jax.lax.broadcasted_iota(jnp.int32, (C, HW), 1)
    wv = p & (W - 1)

    x32 = xb.astype(jnp.float32)
    xm = jnp.where(wv == 0, 0.0, pltpu.roll(x32, 1, axis=1)).astype(jnp.bfloat16)
    xp = jnp.where(wv == W - 1, 0.0,
                   pltpu.roll(x32, HW - 1, axis=1)).astype(jnp.bfloat16)
    im3 = jnp.concatenate([xm, xb, xp], axis=0)

    Y = jnp.dot(w_ref[1], im3, preferred_element_type=jnp.float32)
    d0 = jnp.dot(w_ref[0], im3, preferred_element_type=jnp.float32)
    Y = Y + jnp.where(p >= W, pltpu.roll(d0, W, axis=1), 0.0)
    d2 = jnp.dot(w_ref[2], im3, preferred_element_type=jnp.float32)
    Y = Y + jnp.where(p < HW - W, pltpu.roll(d2, HW - W, axis=1), 0.0)

    rsum = jnp.sum(Y, axis=1, keepdims=True)
    rsq = jnp.sum(Y * Y, axis=1, keepdims=True)
    gm = gmat_ref[...]
    mean = jnp.dot(gm, rsum, preferred_element_type=jnp.float32)
    msq = jnp.dot(gm, rsq, preferred_element_type=jnp.float32)
    var = msq - mean * mean
    a = jax.lax.rsqrt(var + EPS) * scale_ref[...]
    b = bias_ref[...] - mean * a
    act = _gelu(Y * a + b)
    if res_ref is not None:
        act = act + res_ref[...].astype(jnp.float32)
    o_ref[...] = act.astype(o_ref.dtype)


def _mk_conv_body(with_res, img_w):
    if with_res:
        def body(x_ref, w_ref, scale_ref, bias_ref, gmat_ref, res_ref, o_ref):
            _conv_kernel(x_ref, w_ref, scale_ref, bias_ref, gmat_ref, res_ref,
                         o_ref, img_w)
    else:
        def body(x_ref, w_ref, scale_ref, bias_ref, gmat_ref, o_ref):
            _conv_kernel(x_ref, w_ref, scale_ref, bias_ref, gmat_ref, None,
                         o_ref, img_w)
    return body


def _conv_stage(x_col, w4, scale, bias, gmat, img_w, img_h,
                res_col=None, out_f32=False):
    C, N = x_col.shape
    HW = img_h * img_w
    B = N // HW
    kb = pl.BlockSpec((C, HW), lambda b: (0, b))
    wspec = pl.BlockSpec(w4.shape, lambda b: (0, 0, 0))
    cspec = pl.BlockSpec((C, 1), lambda b: (0, 0))
    gspec = pl.BlockSpec((C, C), lambda b: (0, 0))
    args = [x_col, w4, scale, bias, gmat]
    specs = [kb, wspec, cspec, cspec, gspec]
    if res_col is not None:
        args.append(res_col)
        specs.append(kb)
    return pl.pallas_call(
        _mk_conv_body(res_col is not None, img_w),
        out_shape=jax.ShapeDtypeStruct((C, N), jnp.float32 if out_f32 else jnp.bfloat16),
        grid=(B,),
        in_specs=specs,
        out_specs=kb,
        compiler_params=pltpu.CompilerParams(
            dimension_semantics=("arbitrary",),
            vmem_limit_bytes=100 * 1024 * 1024,
        ),
        name="conv_gn_act",
    )(*args)


def kernel(temporal_features, sup, params):
    B, T, C, H, W = temporal_features.shape
    N = B * H * W
    HID = params['blocks'][0]['ffn_w1'].shape[0]

    sup_hw = jax.image.resize(sup, (B, T, H, W), 'bilinear')
    x_cm = temporal_features.transpose(1, 2, 0, 3, 4).reshape(T, C, N).astype(
        jnp.bfloat16)
    sup_tn = sup_hw.transpose(1, 0, 2, 3).reshape(T, N)

    bf = lambda a: a.astype(jnp.bfloat16)
    col = lambda v: v.reshape(-1, 1).astype(jnp.float32)
    p1, p2 = params['blocks']
    merged = jnp.dot(p2['in_w'], p1['out_w'])
    ws = [bf(p1['in_w']), bf(p1['ffn_w1']), col(p1['ffn_b1']),
          bf(p1['ffn_w2']), col(p1['ffn_b2']),
          bf(p1['gate_w']), col(p1['gate_b']),
          col(p1['sup_w'][:, 0]), col(p1['sup_b']), bf(merged),
          bf(p2['ffn_w1']), col(p2['ffn_b1']),
          bf(p2['ffn_w2']), col(p2['ffn_b2']),
          bf(p2['gate_w']), col(p2['gate_b']),
          col(p2['sup_w'][:, 0]), col(p2['sup_b']),
          bf(p2['out_w'] * (1.0 / T))]

    nb = min(_NB, N)
    nblk = N // nb
    ncore = 2 if nblk % 2 == 0 else 1
    nseq = nblk // ncore
    wspecs = [pl.BlockSpec(w.shape, lambda i, j: (0, 0)) for w in ws]

    y_col = pl.pallas_call(
        _ssm_kernel,
        out_shape=jax.ShapeDtypeStruct((C, N), jnp.bfloat16),
        grid=(ncore, nseq),
        in_specs=[
            pl.BlockSpec((T, C, nb), lambda i, j: (0, 0, i * nseq + j)),
            pl.BlockSpec((T, nb), lambda i, j: (0, i * nseq + j)),
        ] + wspecs,
        out_specs=pl.BlockSpec((C, nb), lambda i, j: (0, i * nseq + j)),
        compiler_params=pltpu.CompilerParams(
            dimension_semantics=("arbitrary", "arbitrary"),
            vmem_limit_bytes=100 * 1024 * 1024,
        ),
        name="ssm_fused",
    )(x_cm, sup_tn, *ws)

    cg = C // GROUPS
    gi = jnp.arange(C) // cg
    gmat = (gi[:, None] == gi[None, :]).astype(jnp.float32) / (H * W * cg)

    y1 = _conv_stage(y_col, bf(params['post_conv_w'].transpose(2, 0, 3, 1).reshape(3, C, 3 * C)),
                     col(params['post_gn_s']), col(params['post_gn_b']), gmat,
                     W, H)
    r1 = _conv_stage(y1, bf(params['rc1_w'].transpose(2, 0, 3, 1).reshape(3, C, 3 * C)),
                     col(params['rg1_s']), col(params['rg1_b']), gmat, W, H)
    out_col = _conv_stage(r1, bf(params['rc2_w'].transpose(2, 0, 3, 1).reshape(3, C, 3 * C)),
                          col(params['rg2_s']), col(params['rg2_b']), gmat,
                          W, H, res_col=y1)

    return out_col.reshape(C, B, H, W).transpose(1, 0, 2, 3).astype(jnp.float32)


# speedup vs baseline: 4.5608x; 1.0214x over previous
import jax
import jax.numpy as jnp
from jax.experimental import pallas as pl
from jax.experimental.pallas import tpu as pltpu

GROUPS = 16
EPS = 1e-5
_NB = 1024


def _gelu(x):
    return 0.5 * x * (1.0 + jax.lax.erf(x * (2.0 ** -0.5)))


def _ssm_kernel(x_ref, sup_ref,
                in1, w11, b11, w12, b12, g1w, g1b, s1w, s1b, o1w,
                w21, b21, w22, b22, g2w, g2b, s2w, s2b, o2w,
                o_ref):
    T, C, NB = x_ref.shape

    sup_rows = [sup_ref[t:t + 1, :] for t in range(T)]

    def dense(Xb, inw, w1, bias1, w2, bias2, gw, gbias, colscale=None):
        Xin = jnp.dot(inw[...], Xb, preferred_element_type=jnp.float32)
        if colscale is not None:
            Xin = Xin * colscale
        hh = _gelu(jnp.dot(w1[...], Xin.astype(jnp.bfloat16),
                           preferred_element_type=jnp.float32) + b1_ld(bias1))
        h = jnp.dot(w2[...], hh.astype(jnp.bfloat16),
                    preferred_element_type=jnp.float32) + b1_ld(bias2)
        Xt = Xin + h
        G = jax.nn.sigmoid(jnp.dot(gw[...], Xt.astype(jnp.bfloat16),
                                   preferred_element_type=jnp.float32) + b1_ld(gbias))
        return Xt, G

    def b1_ld(ref):
        return ref[...]

    def scan(Xt, G, sw, sb, accumulate):
        s = jnp.zeros((C, NB), jnp.float32)
        outs = []
        ssum = jnp.zeros((C, NB), jnp.float32)
        swv = sw[...]
        sbv = sb[...]
        sup_cat = jnp.concatenate(sup_rows, axis=1)
        Gs = G * jax.nn.sigmoid(swv * sup_cat + sbv)
        for t in range(T):
            g = Gs[:, t * NB:(t + 1) * NB]
            x_t = Xt[:, t * NB:(t + 1) * NB]
            s = s + g * (x_t - s)
            if accumulate:
                ssum = ssum + s
            else:
                outs.append(s)
        return outs, ssum

    X = jnp.concatenate([x_ref[t] for t in range(T)], axis=1)
    fac = 1.0 - jnp.concatenate(sup_rows, axis=1)

    Xt1, G1 = dense(X, in1, w11, b11, w12, b12, g1w, g1b, colscale=fac)
    S1, _ = scan(Xt1, G1, s1w, s1b, accumulate=False)
    S1cat = jnp.concatenate(S1, axis=1)

    Xt2, G2 = dense(S1cat.astype(jnp.bfloat16), o1w, w21, b21, w22, b22,
                    g2w, g2b)
    _, ssum = scan(Xt2, G2, s2w, s2b, accumulate=True)

    o_ref[...] = jnp.dot(o2w[...], ssum.astype(jnp.bfloat16),
                         preferred_element_type=jnp.float32).astype(jnp.bfloat16)


def _conv_kernel(x_ref, w_ref, scale_ref, bias_ref, gmat_ref, res_ref, o_ref, W):
    C, HW = x_ref.shape
    xb = x_ref[...]
    p = jax.lax.broadcasted_iota(jnp.int32, (C, HW), 1)
    wv = p & (W - 1)

    x32 = xb.astype(jnp.float32)
    xm = jnp.where(wv == 0, 0.0, pltpu.roll(x32, 1, axis=1)).astype(jnp.bfloat16)
    xp = jnp.where(wv == W - 1, 0.0,
                   pltpu.roll(x32, HW - 1, axis=1)).astype(jnp.bfloat16)
    im3 = jnp.concatenate([xm, xb, xp], axis=0)

    Y = jnp.dot(w_ref[1], im3, preferred_element_type=jnp.float32)
    d0 = jnp.dot(w_ref[0], im3, preferred_element_type=jnp.float32)
    Y = Y + jnp.where(p >= W, pltpu.roll(d0, W, axis=1), 0.0)
    d2 = jnp.dot(w_ref[2], im3, preferred_element_type=jnp.float32)
    Y = Y + jnp.where(p < HW - W, pltpu.roll(d2, HW - W, axis=1), 0.0)

    rsum = jnp.sum(Y, axis=1, keepdims=True)
    rsq = jnp.sum(Y * Y, axis=1, keepdims=True)
    gm = gmat_ref[...]
    mean = jnp.dot(gm, rsum, preferred_element_type=jnp.float32)
    msq = jnp.dot(gm, rsq, preferred_element_type=jnp.float32)
    var = msq - mean * mean
    a = jax.lax.rsqrt(var + EPS) * scale_ref[...]
    b = bias_ref[...] - mean * a
    act = _gelu(Y * a + b)
    if res_ref is not None:
        act = act + res_ref[...].astype(jnp.float32)
    o_ref[...] = act.astype(o_ref.dtype)


def _mk_conv_body(with_res, img_w):
    if with_res:
        def body(x_ref, w_ref, scale_ref, bias_ref, gmat_ref, res_ref, o_ref):
            _conv_kernel(x_ref, w_ref, scale_ref, bias_ref, gmat_ref, res_ref,
                         o_ref, img_w)
    else:
        def body(x_ref, w_ref, scale_ref, bias_ref, gmat_ref, o_ref):
            _conv_kernel(x_ref, w_ref, scale_ref, bias_ref, gmat_ref, None,
                         o_ref, img_w)
    return body


def _conv_stage(x_col, w4, scale, bias, gmat, img_w, img_h,
                res_col=None, out_f32=False):
    C, N = x_col.shape
    HW = img_h * img_w
    B = N // HW
    kb = pl.BlockSpec((C, HW), lambda b: (0, b))
    wspec = pl.BlockSpec(w4.shape, lambda b: (0, 0, 0))
    cspec = pl.BlockSpec((C, 1), lambda b: (0, 0))
    gspec = pl.BlockSpec((C, C), lambda b: (0, 0))
    args = [x_col, w4, scale, bias, gmat]
    specs = [kb, wspec, cspec, cspec, gspec]
    if res_col is not None:
        args.append(res_col)
        specs.append(kb)
    return pl.pallas_call(
        _mk_conv_body(res_col is not None, img_w),
        out_shape=jax.ShapeDtypeStruct((C, N), jnp.float32 if out_f32 else jnp.bfloat16),
        grid=(B,),
        in_specs=specs,
        out_specs=kb,
        compiler_params=pltpu.CompilerParams(
            dimension_semantics=("arbitrary",),
            vmem_limit_bytes=100 * 1024 * 1024,
        ),
        name="conv_gn_act",
    )(*args)


def kernel(temporal_features, sup, params):
    B, T, C, H, W = temporal_features.shape
    N = B * H * W
    HID = params['blocks'][0]['ffn_w1'].shape[0]

    sup_hw = jax.image.resize(sup, (B, T, H, W), 'bilinear')
    x_cm = temporal_features.transpose(1, 2, 0, 3, 4).reshape(T, C, N).astype(
        jnp.bfloat16)
    sup_tn = sup_hw.transpose(1, 0, 2, 3).reshape(T, N)

    bf = lambda a: a.astype(jnp.bfloat16)
    col = lambda v: v.reshape(-1, 1).astype(jnp.float32)
    p1, p2 = params['blocks']
    merged = jnp.dot(p2['in_w'], p1['out_w'])
    ws = [bf(p1['in_w']), bf(p1['ffn_w1']), col(p1['ffn_b1']),
          bf(p1['ffn_w2']), col(p1['ffn_b2']),
          bf(p1['gate_w']), col(p1['gate_b']),
          col(p1['sup_w'][:, 0]), col(p1['sup_b']), bf(merged),
          bf(p2['ffn_w1']), col(p2['ffn_b1']),
          bf(p2['ffn_w2']), col(p2['ffn_b2']),
          bf(p2['gate_w']), col(p2['gate_b']),
          col(p2['sup_w'][:, 0]), col(p2['sup_b']),
          bf(p2['out_w'] * (1.0 / T))]

    nb = min(_NB, N)
    nblk = N // nb
    ncore = 2 if nblk % 2 == 0 else 1
    nseq = nblk // ncore
    wspecs = [pl.BlockSpec(w.shape, lambda i, j: (0, 0)) for w in ws]

    y_col = pl.pallas_call(
        _ssm_kernel,
        out_shape=jax.ShapeDtypeStruct((C, N), jnp.bfloat16),
        grid=(ncore, nseq),
        in_specs=[
            pl.BlockSpec((T, C, nb), lambda i, j: (0, 0, i * nseq + j)),
            pl.BlockSpec((T, nb), lambda i, j: (0, i * nseq + j)),
        ] + wspecs,
        out_specs=pl.BlockSpec((C, nb), lambda i, j: (0, i * nseq + j)),
        compiler_params=pltpu.CompilerParams(
            dimension_semantics=("arbitrary", "arbitrary"),
            vmem_limit_bytes=100 * 1024 * 1024,
        ),
        name="ssm_fused",
    )(x_cm, sup_tn, *ws)

    cg = C // GROUPS
    gi = jnp.arange(C) // cg
    gmat = (gi[:, None] == gi[None, :]).astype(jnp.float32) / (H * W * cg)

    y1 = _conv_stage(y_col, bf(params['post_conv_w'].transpose(2, 0, 3, 1).reshape(3, C, 3 * C)),
                     col(params['post_gn_s']), col(params['post_gn_b']), gmat,
                     W, H)
    r1 = _conv_stage(y1, bf(params['rc1_w'].transpose(2, 0, 3, 1).reshape(3, C, 3 * C)),
                     col(params['rg1_s']), col(params['rg1_b']), gmat, W, H)
    out_col = _conv_stage(r1, bf(params['rc2_w'].transpose(2, 0, 3, 1).reshape(3, C, 3 * C)),
                          col(params['rg2_s']), col(params['rg2_b']), gmat,
                          W, H, res_col=y1)

    return out_col.reshape(C, B, H, W).transpose(1, 0, 2, 3).astype(jnp.float32)


# speedup vs baseline: 5.0583x; 1.1091x over previous
import jax
import jax.numpy as jnp
from jax.experimental import pallas as pl
from jax.experimental.pallas import tpu as pltpu

GROUPS = 16
EPS = 1e-5
_NB = 1024


def _gelu(x):
    return 0.5 * x * (1.0 + jax.lax.erf(x * (2.0 ** -0.5)))


def _ssm_kernel(x_ref, sup_ref,
                in1, w11, b11, w12, b12, g1w, g1b, s1w, s1b, o1w,
                w21, b21, w22, b22, g2w, g2b, s2w, s2b, o2w,
                o_ref):
    _, T, C, NB = x_ref.shape

    sup_rows = [sup_ref[t:t + 1, :] for t in range(T)]

    def dense(Xb, inw, w1, bias1, w2, bias2, gw, gbias, colscale=None):
        Xin = jnp.dot(inw[...], Xb, preferred_element_type=jnp.float32)
        if colscale is not None:
            Xin = Xin * colscale
        hh = _gelu(jnp.dot(w1[...], Xin.astype(jnp.bfloat16),
                           preferred_element_type=jnp.float32) + b1_ld(bias1))
        h = jnp.dot(w2[...], hh.astype(jnp.bfloat16),
                    preferred_element_type=jnp.float32) + b1_ld(bias2)
        Xt = Xin + h
        G = jax.nn.sigmoid(jnp.dot(gw[...], Xt.astype(jnp.bfloat16),
                                   preferred_element_type=jnp.float32) + b1_ld(gbias))
        return Xt, G

    def b1_ld(ref):
        return ref[...]

    def scan(Xt, G, sw, sb, accumulate):
        s = jnp.zeros((C, NB), jnp.float32)
        outs = []
        ssum = jnp.zeros((C, NB), jnp.float32)
        swv = sw[...]
        sbv = sb[...]
        sup_cat = jnp.concatenate(sup_rows, axis=1)
        Gs = G * jax.nn.sigmoid(swv * sup_cat + sbv)
        for t in range(T):
            g = Gs[:, t * NB:(t + 1) * NB]
            x_t = Xt[:, t * NB:(t + 1) * NB]
            s = s + g * (x_t - s)
            if accumulate:
                ssum = ssum + s
            else:
                outs.append(s)
        return outs, ssum

    X = jnp.concatenate([x_ref[0, t] for t in range(T)], axis=1)
    fac = 1.0 - jnp.concatenate(sup_rows, axis=1)

    Xt1, G1 = dense(X.astype(jnp.bfloat16), in1, w11, b11, w12, b12,
                    g1w, g1b, colscale=fac)
    S1, _ = scan(Xt1, G1, s1w, s1b, accumulate=False)
    S1cat = jnp.concatenate(S1, axis=1)

    Xt2, G2 = dense(S1cat.astype(jnp.bfloat16), o1w, w21, b21, w22, b22,
                    g2w, g2b)
    _, ssum = scan(Xt2, G2, s2w, s2b, accumulate=True)

    o_ref[...] = jnp.dot(o2w[...], ssum.astype(jnp.bfloat16),
                         preferred_element_type=jnp.float32).astype(jnp.bfloat16)


def _conv_kernel(x_ref, w_ref, scale_ref, bias_ref, gmat_ref, res_ref, o_ref, W):
    C, HW = x_ref.shape
    xb = x_ref[...]
    p = jax.lax.broadcasted_iota(jnp.int32, (C, HW), 1)
    wv = p & (W - 1)

    x32 = xb.astype(jnp.float32)
    xm = jnp.where(wv == 0, 0.0, pltpu.roll(x32, 1, axis=1)).astype(jnp.bfloat16)
    xp = jnp.where(wv == W - 1, 0.0,
                   pltpu.roll(x32, HW - 1, axis=1)).astype(jnp.bfloat16)
    im3 = jnp.concatenate([xm, xb, xp], axis=0)

    Y = jnp.dot(w_ref[1], im3, preferred_element_type=jnp.float32)
    d0 = jnp.dot(w_ref[0], im3, preferred_element_type=jnp.float32)
    Y = Y + jnp.where(p >= W, pltpu.roll(d0, W, axis=1), 0.0)
    d2 = jnp.dot(w_ref[2], im3, preferred_element_type=jnp.float32)
    Y = Y + jnp.where(p < HW - W, pltpu.roll(d2, HW - W, axis=1), 0.0)

    rsum = jnp.sum(Y, axis=1, keepdims=True)
    rsq = jnp.sum(Y * Y, axis=1, keepdims=True)
    gm = gmat_ref[...]
    mean = jnp.dot(gm, rsum, preferred_element_type=jnp.float32)
    msq = jnp.dot(gm, rsq, preferred_element_type=jnp.float32)
    var = msq - mean * mean
    a = jax.lax.rsqrt(var + EPS) * scale_ref[...]
    b = bias_ref[...] - mean * a
    act = _gelu(Y * a + b)
    if res_ref is not None:
        act = act + res_ref[...].astype(jnp.float32)
    if o_ref.ndim == 3:
        o_ref[0] = act.astype(o_ref.dtype)
    else:
        o_ref[...] = act.astype(o_ref.dtype)


def _mk_conv_body(with_res, img_w):
    if with_res:
        def body(x_ref, w_ref, scale_ref, bias_ref, gmat_ref, res_ref, o_ref):
            _conv_kernel(x_ref, w_ref, scale_ref, bias_ref, gmat_ref, res_ref,
                         o_ref, img_w)
    else:
        def body(x_ref, w_ref, scale_ref, bias_ref, gmat_ref, o_ref):
            _conv_kernel(x_ref, w_ref, scale_ref, bias_ref, gmat_ref, None,
                         o_ref, img_w)
    return body


def _conv_stage(x_col, w4, scale, bias, gmat, img_w, img_h,
                res_col=None, out_f32=False):
    C, N = x_col.shape
    HW = img_h * img_w
    B = N // HW
    kb = pl.BlockSpec((C, HW), lambda b: (0, b))
    wspec = pl.BlockSpec(w4.shape, lambda b: (0, 0, 0))
    cspec = pl.BlockSpec((C, 1), lambda b: (0, 0))
    gspec = pl.BlockSpec((C, C), lambda b: (0, 0))
    args = [x_col, w4, scale, bias, gmat]
    specs = [kb, wspec, cspec, cspec, gspec]
    if res_col is not None:
        args.append(res_col)
        specs.append(kb)
    if out_f32:
        out_shape = jax.ShapeDtypeStruct((B, C, HW), jnp.float32)
        out_spec = pl.BlockSpec((1, C, HW), lambda b: (b, 0, 0))
    else:
        out_shape = jax.ShapeDtypeStruct((C, N), jnp.bfloat16)
        out_spec = kb
    return pl.pallas_call(
        _mk_conv_body(res_col is not None, img_w),
        out_shape=out_shape,
        grid=(B,),
        in_specs=specs,
        out_specs=out_spec,
        compiler_params=pltpu.CompilerParams(
            dimension_semantics=("arbitrary",),
            vmem_limit_bytes=100 * 1024 * 1024,
        ),
        name="conv_gn_act",
    )(*args)


def kernel(temporal_features, sup, params):
    B, T, C, H, W = temporal_features.shape
    N = B * H * W
    HID = params['blocks'][0]['ffn_w1'].shape[0]

    sup_hw = jax.image.resize(sup, (B, T, H, W), 'bilinear')
    xr = temporal_features.reshape(B, T, C, H * W)
    sup_tn = sup_hw.transpose(1, 0, 2, 3).reshape(T, N)

    bf = lambda a: a.astype(jnp.bfloat16)
    col = lambda v: v.reshape(-1, 1).astype(jnp.float32)
    p1, p2 = params['blocks']
    merged = jnp.dot(p2['in_w'], p1['out_w'])
    ws = [bf(p1['in_w']), bf(p1['ffn_w1']), col(p1['ffn_b1']),
          bf(p1['ffn_w2']), col(p1['ffn_b2']),
          bf(p1['gate_w']), col(p1['gate_b']),
          col(p1['sup_w'][:, 0]), col(p1['sup_b']), bf(merged),
          bf(p2['ffn_w1']), col(p2['ffn_b1']),
          bf(p2['ffn_w2']), col(p2['ffn_b2']),
          bf(p2['gate_w']), col(p2['gate_b']),
          col(p2['sup_w'][:, 0]), col(p2['sup_b']),
          bf(p2['out_w'] * (1.0 / T))]

    HW = H * W
    nb = min(_NB, HW)
    nseq = HW // nb
    wspecs = [pl.BlockSpec(w.shape, lambda i, j: (0, 0)) for w in ws]

    y_col = pl.pallas_call(
        _ssm_kernel,
        out_shape=jax.ShapeDtypeStruct((C, N), jnp.bfloat16),
        grid=(B, nseq),
        in_specs=[
            pl.BlockSpec((1, T, C, nb), lambda i, j: (i, 0, 0, j)),
            pl.BlockSpec((T, nb), lambda i, j: (0, i * nseq + j)),
        ] + wspecs,
        out_specs=pl.BlockSpec((C, nb), lambda i, j: (0, i * nseq + j)),
        compiler_params=pltpu.CompilerParams(
            dimension_semantics=("arbitrary", "arbitrary"),
            vmem_limit_bytes=100 * 1024 * 1024,
        ),
        name="ssm_fused",
    )(xr, sup_tn, *ws)

    cg = C // GROUPS
    gi = jnp.arange(C) // cg
    gmat = (gi[:, None] == gi[None, :]).astype(jnp.float32) / (H * W * cg)

    y1 = _conv_stage(y_col, bf(params['post_conv_w'].transpose(2, 0, 3, 1).reshape(3, C, 3 * C)),
                     col(params['post_gn_s']), col(params['post_gn_b']), gmat,
                     W, H)
    r1 = _conv_stage(y1, bf(params['rc1_w'].transpose(2, 0, 3, 1).reshape(3, C, 3 * C)),
                     col(params['rg1_s']), col(params['rg1_b']), gmat, W, H)
    out_bchw = _conv_stage(r1, bf(params['rc2_w'].transpose(2, 0, 3, 1).reshape(3, C, 3 * C)),
                           col(params['rg2_s']), col(params['rg2_b']), gmat,
                           W, H, res_col=y1, out_f32=True)

    return out_bchw.reshape(B, C, H, W)
